# Initial kernel scaffold; baseline (speedup 1.0000x reference)
import jax
import jax.numpy as jnp
from jax.experimental import pallas as pl

_NOUT = 1024
_R1 = 0.5
_R2 = 0.5
_K1 = 3
_K2 = 2


def _dists_mm(points1, points2):
    B, M, C = points1.shape
    N = points2.shape[1]
    d = jnp.sum(points1 ** 2, axis=-1).reshape(B, M, 1) + jnp.sum(points2 ** 2, axis=-1).reshape(B, 1, N)
    d = d - 2.0 * jnp.matmul(points1, jnp.swapaxes(points2, 1, 2))
    d = jnp.where(d < 0, jnp.full_like(d, 1e-07), d)
    return jnp.sqrt(d).astype(jnp.float32)


def _dists_ma(points1, points2):
    B, M, C = points1.shape
    N = points2.shape[1]
    n1 = jnp.sum(points1 ** 2, axis=-1).reshape(B, M, 1)
    n2 = jnp.sum(points2 ** 2, axis=-1).reshape(B, 1, N)
    x1 = points1[:, :, 0:1]; y1 = points1[:, :, 1:2]; z1 = points1[:, :, 2:3]
    x2 = points2[:, :, 0].reshape(B, 1, N)
    y2 = points2[:, :, 1].reshape(B, 1, N)
    z2 = points2[:, :, 2].reshape(B, 1, N)
    dot = (x1 * x2 + y1 * y2) + z1 * z2
    d = n1 + n2
    d = d - 2.0 * dot
    d = jnp.where(d < 0, jnp.full_like(d, 1e-07), d)
    return jnp.sqrt(d).astype(jnp.float32)


def _ball_query(xyz, new_xyz, radius, K, dfn):
    B, N, _ = xyz.shape
    M = new_xyz.shape[1]
    gi = jnp.broadcast_to(jnp.arange(N, dtype=jnp.int32).reshape(1, 1, N), (B, M, N))
    d = dfn(new_xyz, xyz)
    gi = jnp.where(d > radius, N, gi)
    gi = jnp.sort(gi, axis=-1)[:, :, :K]
    gmin = jnp.broadcast_to(gi[:, :, 0:1], gi.shape)
    gi = jnp.where(gi == N, gmin, gi)
    return gi


def _fps(xyz, M, dfn):
    B, N, _ = xyz.shape
    key = jax.random.key(42)
    inds0 = jax.random.randint(key, (B,), 0, N, dtype=jnp.int32)
    dists0 = jnp.full((B, N), 100000.0, dtype=jnp.float32)
    batch = jnp.arange(B)

    def step(carry, _):
        dists, inds = carry
        cur_point = xyz[batch, inds, :]
        cur_dist = dfn(cur_point[:, None, :], xyz)[:, 0, :]
        dists = jnp.minimum(dists, cur_dist)
        new_inds = jnp.argmax(dists, axis=1).astype(jnp.int32)
        return (dists, new_inds), inds

    _, centroids = jax.lax.scan(step, (dists0, inds0), xs=None, length=M)
    return centroids.T


def _gather_points(points, inds):
    B = points.shape[0]
    batch = jnp.arange(B).reshape((B,) + (1,) * (inds.ndim - 1))
    batch = jnp.broadcast_to(batch, inds.shape)
    return points[batch, inds, :]


def _forward(rgb, xyz, para1, para2, para3, dfn):
    fi_id = _fps(jax.lax.stop_gradient(xyz), _NOUT, dfn)
    fi_xyz = _gather_points(xyz, fi_id)
    fij_id = _ball_query(xyz, fi_xyz, _R1, _K1 + 1, dfn)
    fi1_id = fij_id[:, :, 1]
    fi2_id = fij_id[:, :, 2]
    fi3_id = fij_id[:, :, 3]
    sub1 = _gather_points(xyz, fi1_id)
    sub2 = _gather_points(xyz, fi2_id)
    sub3 = _gather_points(xyz, fi3_id)
    sc1_id = _ball_query(xyz, sub1, _R2, _K2 + 1, dfn)
    sc2_id = _ball_query(xyz, sub2, _R2, _K2 + 1, dfn)
    sc3_id = _ball_query(xyz, sub3, _R2, _K2 + 1, dfn)
    rgbT = jnp.swapaxes(rgb, 1, 2)
    subcloud1 = _gather_points(rgbT, sc1_id)
    subcloud2 = _gather_points(rgbT, sc2_id)
    subcloud3 = _gather_points(rgbT, sc3_id)
    y1 = jnp.matmul(subcloud1, para1)
    y1 = y1[:, :, 0:1, :] + y1[:, :, 1:2, :] + y1[:, :, 2:3, :]
    y1 = jax.nn.relu(y1)
    y2 = jnp.matmul(subcloud2, para2)
    y2 = y2[:, :, 0:1, :] + y2[:, :, 1:2, :] + y2[:, :, 2:3, :]
    y2 = jax.nn.relu(y2)
    y3 = jnp.matmul(subcloud3, para3)
    y3 = y3[:, :, 0:1, :] + y3[:, :, 1:2, :] + y3[:, :, 2:3, :]
    y3 = jax.nn.relu(y3)
    y = jnp.concatenate((y1, y2, y3), axis=2)
    y = jnp.max(y, axis=2)
    y = jnp.swapaxes(y, 1, 2)
    return y, fi_id, fi_xyz, (fij_id, sc1_id, sc2_id, sc3_id)


def kernel(rgb, xyz, para1, para2, para3):
    y, fi_id_mm, fi_xyz, ids_mm = _forward(rgb, xyz, para1, para2, para3, _dists_mm)
    _, fi_id_ma, _, ids_ma = _forward(rgb, xyz, para1, para2, para3, _dists_ma)
    flips = jnp.sum((fi_id_mm != fi_id_ma).astype(jnp.int32))
    for a, b in zip(ids_mm, ids_ma):
        flips = flips + jnp.sum((a != b).astype(jnp.int32))
    fi_xyz_diag = fi_xyz + flips.astype(jnp.float32)
    return y, fi_xyz_diag


# speedup vs baseline: 16.5847x; 16.5847x over previous
import functools

import jax
import jax.numpy as jnp
from jax.experimental import pallas as pl

_NOUT = 1024
_R1 = 0.5
_R2 = 0.5


def _rne_bf16(x):
    u = jax.lax.bitcast_convert_type(x, jnp.uint32)
    lsb = jax.lax.shift_right_logical(u, jnp.uint32(16)) & jnp.uint32(1)
    u = (u + jnp.uint32(0x7FFF) + lsb) & jnp.uint32(0xFFFF0000)
    return jax.lax.bitcast_convert_type(u, jnp.float32)


def _to_bf16(x):
    return _rne_bf16(x).astype(jnp.bfloat16)


def _first_k(mask, lane, n, k):
    out = []
    m = mask
    for _ in range(k):
        i = jnp.min(jnp.where(m, lane, n), axis=1, keepdims=True)
        out.append(i)
        m = m & (lane > i)
    return out


def _gather_rows(sel, v):
    return jnp.sum(jnp.where(sel, v, 0.0), axis=1, keepdims=True)


def _centers(idx, lane, X, Y, Z):
    sel = lane == idx
    return _gather_rows(sel, X), _gather_rows(sel, Y), _gather_rows(sel, Z)


def _dist_row(cx, cy, cz, xyztb, n2, n):
    n1 = (cx * cx + cy * cy) + cz * cz
    c = jnp.concatenate([cx, cy, cz], axis=1)
    cb = _to_bf16(c)
    dot = jax.lax.dot_general(cb, xyztb, (((1,), (0,)), ((), ())),
                              preferred_element_type=jnp.float32)
    d2 = (n1 + n2) - 2.0 * dot
    d2 = jnp.where(d2 < 0, jnp.full_like(d2, 1e-07), d2)
    return jnp.sqrt(d2)


def _fps_kernel(x_ref, y_ref, z_ref, stk_ref, i0_ref, out_ref, *, n, m, b):
    X = x_ref[...]
    Y = y_ref[...]
    Z = z_ref[...]
    stk = stk_ref[...]
    n2 = (X * X + Y * Y) + Z * Z
    lane = jax.lax.broadcasted_iota(jnp.int32, (b, n), 1)
    mlane = jax.lax.broadcasted_iota(jnp.int32, (b, m), 1)
    col = jax.lax.broadcasted_iota(jnp.int32, (b, 4 * b), 1)
    row4 = 4 * jax.lax.broadcasted_iota(jnp.int32, (b, 4 * b), 0)
    ind0 = i0_ref[...]
    dists0 = jnp.full((b, n), 100000.0, dtype=jnp.float32)

    def step(t, carry):
        dists, ind = carry
        out_ref[...] = jnp.where(mlane == t, jnp.broadcast_to(ind, (b, m)),
                                 out_ref[...])
        sel = lane == ind
        cx = _gather_rows(sel, X)
        cy = _gather_rows(sel, Y)
        cz = _gather_rows(sel, Z)
        n1 = (cx * cx + cy * cy) + cz * cz
        curm = (jnp.where(col == row4, cx, 0.0) +
                jnp.where(col == row4 + 1, cy, 0.0) +
                jnp.where(col == row4 + 2, cz, 0.0))
        dot = jax.lax.dot_general(_to_bf16(curm), stk,
                                  (((1,), (0,)), ((), ())),
                                  preferred_element_type=jnp.float32)
        d2 = (n1 + n2) - 2.0 * dot
        d2 = jnp.where(d2 < 0, jnp.full_like(d2, 1e-07), d2)
        d = jnp.sqrt(d2)
        dists = jnp.minimum(dists, d)
        mx = jnp.max(dists, axis=1, keepdims=True)
        ind_new = jnp.min(jnp.where(dists == mx, lane, n), axis=1,
                          keepdims=True)
        return dists, ind_new

    jax.lax.fori_loop(0, m, step, (dists0, ind0))


def _qa_kernel(idx_ref, xyzt_ref, xyztb_ref, fxyz_ref, f1_ref, f2_ref,
               f3_ref, *, n, blk):
    idx = idx_ref[0]
    xyz_t = xyzt_ref[0]
    xyztb = xyztb_ref[0]
    X = xyz_t[0:1, :]
    Y = xyz_t[1:2, :]
    Z = xyz_t[2:3, :]
    n2 = (X * X + Y * Y) + Z * Z
    lane = jax.lax.broadcasted_iota(jnp.int32, (blk, n), 1)
    cx, cy, cz = _centers(idx, lane, X, Y, Z)
    d = _dist_row(cx, cy, cz, xyztb, n2, n)
    mask = d <= _R1
    i1, i2, i3, i4 = _first_k(mask, lane, n, 4)
    i2 = jnp.where(i2 == n, i1, i2)
    i3 = jnp.where(i3 == n, i1, i3)
    i4 = jnp.where(i4 == n, i1, i4)
    fxyz_ref[0] = jnp.concatenate([cx, cy, cz], axis=1)
    f1_ref[0] = i2
    f2_ref[0] = i3
    f3_ref[0] = i4


def _r_kernel(rgb_ref, p1_ref, p2_ref, p3_ref, r1_ref, r2_ref, r3_ref):
    rgb = rgb_ref[0]
    for p_ref, r_ref in ((p1_ref, r1_ref), (p2_ref, r2_ref),
                         (p3_ref, r3_ref)):
        r_ref[0] = jax.lax.dot_general(
            rgb, p_ref[...], (((0,), (0,)), ((), ())),
            preferred_element_type=jnp.float32)


def _qb_kernel(i1_ref, i2_ref, i3_ref, xyzt_ref, xyztb_ref, r1_ref, r2_ref,
               r3_ref, y_ref, *, n, blk):
    xyz_t = xyzt_ref[0]
    xyztb = xyztb_ref[0]
    X = xyz_t[0:1, :]
    Y = xyz_t[1:2, :]
    Z = xyz_t[2:3, :]
    n2 = (X * X + Y * Y) + Z * Z
    lane = jax.lax.broadcasted_iota(jnp.int32, (blk, n), 1)
    ym = None
    for idx_ref, r_ref in ((i1_ref, r1_ref), (i2_ref, r2_ref),
                           (i3_ref, r3_ref)):
        idx = idx_ref[0]
        cx, cy, cz = _centers(idx, lane, X, Y, Z)
        d = _dist_row(cx, cy, cz, xyztb, n2, n)
        mask = d <= _R2
        j1, j2, j3 = _first_k(mask, lane, n, 3)
        j2 = jnp.where(j2 == n, j1, j2)
        j3 = jnp.where(j3 == n, j1, j3)
        w = ((lane == j1).astype(jnp.float32) +
             (lane == j2).astype(jnp.float32) +
             (lane == j3).astype(jnp.float32))
        s = jax.lax.dot_general(w, r_ref[0], (((1,), (0,)), ((), ())),
                                preferred_element_type=jnp.float32)
        z = jnp.maximum(s, 0.0)
        ym = z if ym is None else jnp.maximum(ym, z)
    y_ref[0] = ym


def _interp():
    return False


def kernel(rgb, xyz, para1, para2, para3):
    B, FIN, N = rgb.shape
    M = _NOUT
    FOUT = para1.shape[1]
    BLK = 256
    interpret = _interp()

    X = xyz[:, :, 0]
    Y = xyz[:, :, 1]
    Z = xyz[:, :, 2]
    xyz_t = jnp.swapaxes(xyz, 1, 2)
    xyz_tb = xyz_t.astype(jnp.bfloat16)
    stack = jnp.pad(xyz_t, ((0, 0), (0, 1), (0, 0))).reshape(4 * B, N)
    stack_b = stack.astype(jnp.bfloat16)
    inds0 = jax.random.randint(jax.random.key(42), (B,), 0, N,
                               dtype=jnp.int32).reshape(B, 1)

    fi_id = pl.pallas_call(
        functools.partial(_fps_kernel, n=N, m=M, b=B),
        out_shape=jax.ShapeDtypeStruct((B, M), jnp.int32),
        interpret=interpret,
    )(X, Y, Z, stack_b, inds0)

    fi_id3 = fi_id.reshape(B, M, 1)

    nblk = M // BLK
    fi_xyz, f1, f2, f3 = pl.pallas_call(
        functools.partial(_qa_kernel, n=N, blk=BLK),
        grid=(B, nblk),
        in_specs=[
            pl.BlockSpec((1, BLK, 1), lambda b, mb: (b, mb, 0)),
            pl.BlockSpec((1, 3, N), lambda b, mb: (b, 0, 0)),
            pl.BlockSpec((1, 3, N), lambda b, mb: (b, 0, 0)),
        ],
        out_specs=[
            pl.BlockSpec((1, BLK, 3), lambda b, mb: (b, mb, 0)),
            pl.BlockSpec((1, BLK, 1), lambda b, mb: (b, mb, 0)),
            pl.BlockSpec((1, BLK, 1), lambda b, mb: (b, mb, 0)),
            pl.BlockSpec((1, BLK, 1), lambda b, mb: (b, mb, 0)),
        ],
        out_shape=[
            jax.ShapeDtypeStruct((B, M, 3), jnp.float32),
            jax.ShapeDtypeStruct((B, M, 1), jnp.int32),
            jax.ShapeDtypeStruct((B, M, 1), jnp.int32),
            jax.ShapeDtypeStruct((B, M, 1), jnp.int32),
        ],
        interpret=interpret,
    )(fi_id3, xyz_t, xyz_tb)

    r1, r2, r3 = pl.pallas_call(
        _r_kernel,
        grid=(B,),
        in_specs=[
            pl.BlockSpec((1, FIN, N), lambda b: (b, 0, 0)),
            pl.BlockSpec((FIN, FOUT), lambda b: (0, 0)),
            pl.BlockSpec((FIN, FOUT), lambda b: (0, 0)),
            pl.BlockSpec((FIN, FOUT), lambda b: (0, 0)),
        ],
        out_specs=[
            pl.BlockSpec((1, N, FOUT), lambda b: (b, 0, 0)),
            pl.BlockSpec((1, N, FOUT), lambda b: (b, 0, 0)),
            pl.BlockSpec((1, N, FOUT), lambda b: (b, 0, 0)),
        ],
        out_shape=[
            jax.ShapeDtypeStruct((B, N, FOUT), jnp.float32),
            jax.ShapeDtypeStruct((B, N, FOUT), jnp.float32),
            jax.ShapeDtypeStruct((B, N, FOUT), jnp.float32),
        ],
        interpret=interpret,
    )(rgb, para1, para2, para3)

    y = pl.pallas_call(
        functools.partial(_qb_kernel, n=N, blk=BLK),
        grid=(B, nblk),
        in_specs=[
            pl.BlockSpec((1, BLK, 1), lambda b, mb: (b, mb, 0)),
            pl.BlockSpec((1, BLK, 1), lambda b, mb: (b, mb, 0)),
            pl.BlockSpec((1, BLK, 1), lambda b, mb: (b, mb, 0)),
            pl.BlockSpec((1, 3, N), lambda b, mb: (b, 0, 0)),
            pl.BlockSpec((1, 3, N), lambda b, mb: (b, 0, 0)),
            pl.BlockSpec((1, N, FOUT), lambda b, mb: (b, 0, 0)),
            pl.BlockSpec((1, N, FOUT), lambda b, mb: (b, 0, 0)),
            pl.BlockSpec((1, N, FOUT), lambda b, mb: (b, 0, 0)),
        ],
        out_specs=pl.BlockSpec((1, BLK, FOUT), lambda b, mb: (b, mb, 0)),
        out_shape=jax.ShapeDtypeStruct((B, M, FOUT), jnp.float32),
        interpret=interpret,
    )(f1, f2, f3, xyz_t, xyz_tb, r1, r2, r3)

    return jnp.swapaxes(y, 1, 2), fi_xyz
